# baseline (speedup 1.0000x reference)
import functools

import jax
import jax.numpy as jnp
from jax import lax
from jax.experimental import pallas as pl
from jax.experimental.pallas import tpu as pltpu
from jax.experimental.pallas import tpu_sc as plsc

N = 10000
E = 160000
H = 256
HH = 128
NC, NS = 2, 16
ND = 10240

_mesh = plsc.VectorSubcoreMesh(core_axis_name="c", subcore_axis_name="s")


EPT_D = E // (NC * NS)


@functools.partial(
    pl.kernel,
    out_type=jax.ShapeDtypeStruct((NC * ND,), jnp.float32),
    mesh=_mesh,
    scratch_types=[
        pltpu.VMEM((EPT_D,), jnp.int32),
        pltpu.VMEM((5008,), jnp.float32),
        pltpu.VMEM((640,), jnp.float32),
        pltpu.VMEM_SHARED((ND,), jnp.float32),
    ],
)
def _sc_degree(dst_hbm, out_hbm, idxv, onesv, zv, acc):
    c = lax.axis_index("c")
    s = lax.axis_index("s")
    for i in range(313):
        onesv[pl.ds(i * 16, 16)] = jnp.ones((16,), jnp.float32)
    for i in range(40):
        zv[pl.ds(i * 16, 16)] = jnp.zeros((16,), jnp.float32)
    pltpu.sync_copy(zv, acc.at[pl.ds(s * 640, 640)])
    plsc.subcore_barrier()
    g = c * NS + s
    pltpu.sync_copy(dst_hbm.at[pl.ds(g * EPT_D, EPT_D)], idxv)
    pltpu.sync_copy(onesv.at[pl.ds(0, EPT_D)], acc.at[idxv], add=True)
    plsc.subcore_barrier()
    pltpu.sync_copy(acc.at[pl.ds(s * 640, 640)],
                    out_hbm.at[pl.ds(c * ND + s * 640, 640)])


CH = 120
NCH = 84
EPAD = CH * NCH
NP = 10240
TRASH = NP - 1
RPT = NP // NS


@functools.partial(
    pl.kernel,
    out_type=jax.ShapeDtypeStruct((NC, NP, HH), jnp.float32),
    mesh=_mesh,
    scratch_types=[
        [pltpu.VMEM((CH,), jnp.int32)] * 4,
        [pltpu.VMEM((CH,), jnp.int32)] * 4,
        [pltpu.VMEM((CH, HH), jnp.float32)] * 3,
        pltpu.VMEM_SHARED((NP, HH), jnp.float32),
        pltpu.SemaphoreType.DMA,
        pltpu.SemaphoreType.DMA,
        pltpu.SemaphoreType.DMA,
    ],
)
def _sc_edge_pass(src2_hbm, dst_hbm, hs_hbm, z_hbm, out_hbm,
                  sv, dv, rows, acc, gsem, isem, ssem):
    c = lax.axis_index("c")
    s = lax.axis_index("s")
    pltpu.sync_copy(z_hbm, acc.at[pl.ds(s * RPT, RPT), :])
    plsc.subcore_barrier()

    def idx_start(g):
        return (
            pltpu.async_copy(
                src2_hbm.at[pl.ds((c * NS + s) * EPAD + g * CH, CH)],
                sv[g % 4], isem),
            pltpu.async_copy(
                dst_hbm.at[pl.ds(s * EPAD + g * CH, CH)], dv[g % 4], isem),
        )

    def gather_start(g):
        return pltpu.async_copy(hs_hbm.at[sv[g % 4]], rows[g % 3], gsem)

    ipend = {g: idx_start(g) for g in range(min(4, NCH))}
    gpend = {}
    for g in range(min(3, NCH)):
        ipend[g][0].wait()
        ipend[g][1].wait()
        gpend[g] = gather_start(g)
    sprev = None
    for g in range(NCH):
        gpend.pop(g).wait()
        scur = pltpu.async_copy(rows[g % 3], acc.at[dv[g % 4]], ssem, add=True)
        scur.wait()
        if g + 3 < NCH:
            h = ipend.pop(g + 3)
            h[0].wait()
            h[1].wait()
            gpend[g + 3] = gather_start(g + 3)
        if g + 4 < NCH:
            ipend[g + 4] = idx_start(g + 4)
        sprev = scur
    del sprev
    plsc.subcore_barrier()
    pltpu.sync_copy(acc.at[pl.ds(s * RPT, RPT), :],
                    out_hbm.at[c, pl.ds(s * RPT, RPT), :])


R = 1000
NB = N // R


def _elu(x):
    return jnp.where(x > 0, x, jnp.exp(jnp.minimum(x, 0.0)) - 1.0)


def _dinv_of(degp_ref):
    deg = degp_ref[:, 0:1] + degp_ref[:, 1:2] + 1.0
    return lax.rsqrt(deg)


def _tc1_body(x_ref, w_ref, b_ref, degp_ref, wc0_ref, bc0_ref,
              x0_ref, hs_ref, sr1_ref):
    x0 = jnp.dot(x_ref[...], w_ref[0], preferred_element_type=jnp.float32)
    x0 = x0 + b_ref[0]
    x0_ref[...] = x0
    dinv = _dinv_of(degp_ref)
    h1 = jnp.dot(x0, wc0_ref[...], preferred_element_type=jnp.float32)
    hs_ref[...] = h1 * dinv
    sr1_ref[...] = h1 * (dinv * dinv) + bc0_ref[...]


def _tc2_body(acca_ref, accb_ref, sr1_ref, h0_ref, degp_ref, wc1_ref, bc1_ref,
              hs_ref, sr2_ref):
    dinv = _dinv_of(degp_ref)
    acc = jnp.concatenate([acca_ref[0], accb_ref[0]], axis=1)
    x1 = _elu(acc * dinv + sr1_ref[...]) + h0_ref[...]
    h2 = jnp.dot(x1, wc1_ref[...], preferred_element_type=jnp.float32)
    hs_ref[...] = h2 * dinv
    sr2_ref[...] = h2 * (dinv * dinv) + bc1_ref[...]


def _tc3_body(acca_ref, accb_ref, sr2_ref, h0_ref, degp_ref, out_ref):
    dinv = _dinv_of(degp_ref)
    acc = jnp.concatenate([acca_ref[0], accb_ref[0]], axis=1)
    out_ref[...] = _elu(acc * dinv + sr2_ref[...]) + h0_ref[...]


def _rb(width):
    return pl.BlockSpec((R, width), lambda i: (i, 0))


def _acc_spec(core):
    return pl.BlockSpec((1, R, HH), lambda i, core=core: (core, i, 0))


def _full(shape):
    return pl.BlockSpec(shape, lambda i: tuple(0 for _ in shape))


_f32 = jnp.float32


def _tc1(x_cat, w_stack, b_stack, degp, wc0, bc0):
    return pl.pallas_call(
        _tc1_body,
        grid=(NB,),
        in_specs=[
            _rb(H),
            pl.BlockSpec((1, H, H), lambda i: (i * R // (N // 2), 0, 0)),
            pl.BlockSpec((1, 1, H), lambda i: (i * R // (N // 2), 0, 0)),
            _rb(2),
            _full((H, H)),
            _full((1, H)),
        ],
        out_specs=[_rb(H), _rb(H), _rb(H)],
        out_shape=[
            jax.ShapeDtypeStruct((N, H), _f32),
            jax.ShapeDtypeStruct((N, H), _f32),
            jax.ShapeDtypeStruct((N, H), _f32),
        ],
    )(x_cat, w_stack, b_stack, degp, wc0, bc0)


def _tc2(acc, sr1, h0, degp, wc1, bc1):
    return pl.pallas_call(
        _tc2_body,
        grid=(NB,),
        in_specs=[
            _acc_spec(0),
            _acc_spec(1),
            _rb(H),
            _rb(H),
            _rb(2),
            _full((H, H)),
            _full((1, H)),
        ],
        out_specs=[_rb(H), _rb(H)],
        out_shape=[
            jax.ShapeDtypeStruct((N, H), _f32),
            jax.ShapeDtypeStruct((N, H), _f32),
        ],
    )(acc, acc, sr1, h0, degp, wc1, bc1)


def _tc3(acc, sr2, h0, degp):
    return pl.pallas_call(
        _tc3_body,
        grid=(NB,),
        in_specs=[
            _acc_spec(0),
            _acc_spec(1),
            _rb(H),
            _rb(H),
            _rb(2),
        ],
        out_specs=_rb(H),
        out_shape=jax.ShapeDtypeStruct((N, H), _f32),
    )(acc, acc, sr2, h0, degp)


def kernel(x_lnc, x_dis, edge_index, W_lnc, b_lnc, W_dis, b_dis,
           W_c0, b_c0, W_c1, b_c1):
    src = edge_index[0]
    dst = edge_index[1]
    srcr = jnp.pad(src.reshape(NS, E // NS), ((0, 0), (0, EPAD - E // NS)))
    dstp = jnp.pad(dst.reshape(NS, E // NS), ((0, 0), (0, EPAD - E // NS)),
                   constant_values=TRASH).reshape(-1)
    src2 = jnp.stack([2 * srcr, 2 * srcr + 1]).reshape(-1)

    deg_flat = _sc_degree(dst)
    degp = deg_flat.reshape(NC, ND).transpose(1, 0)

    x_cat = jnp.concatenate([x_lnc, x_dis], axis=0)
    w_stack = jnp.stack([W_lnc, W_dis])
    b_stack = jnp.stack([b_lnc, b_dis]).reshape(2, 1, H)
    z_rows = jnp.zeros((RPT, HH), _f32)

    h0, hs1, sr1 = _tc1(x_cat, w_stack, b_stack, degp,
                        W_c0, b_c0.reshape(1, H))
    acc1 = _sc_edge_pass(src2, dstp, hs1.reshape(2 * N, HH), z_rows)

    hs2, sr2 = _tc2(acc1, sr1, h0, degp, W_c1, b_c1.reshape(1, H))
    acc2 = _sc_edge_pass(src2, dstp, hs2.reshape(2 * N, HH), z_rows)

    return _tc3(acc2, sr2, h0, degp)


# speedup vs baseline: 1.0009x; 1.0009x over previous
import functools

import jax
import jax.numpy as jnp
from jax import lax
from jax.experimental import pallas as pl
from jax.experimental.pallas import tpu as pltpu
from jax.experimental.pallas import tpu_sc as plsc

N = 10000
E = 160000
H = 256
HH = 128
NC, NS = 2, 16
ND = 10240

_mesh = plsc.VectorSubcoreMesh(core_axis_name="c", subcore_axis_name="s")


EPT_D = E // (NC * NS)


@functools.partial(
    pl.kernel,
    out_type=jax.ShapeDtypeStruct((NC * ND,), jnp.float32),
    mesh=_mesh,
    scratch_types=[
        pltpu.VMEM((EPT_D,), jnp.int32),
        pltpu.VMEM((5008,), jnp.float32),
        pltpu.VMEM((640,), jnp.float32),
        pltpu.VMEM_SHARED((ND,), jnp.float32),
    ],
)
def _sc_degree(dst_hbm, out_hbm, idxv, onesv, zv, acc):
    c = lax.axis_index("c")
    s = lax.axis_index("s")
    for i in range(313):
        onesv[pl.ds(i * 16, 16)] = jnp.ones((16,), jnp.float32)
    for i in range(40):
        zv[pl.ds(i * 16, 16)] = jnp.zeros((16,), jnp.float32)
    pltpu.sync_copy(zv, acc.at[pl.ds(s * 640, 640)])
    plsc.subcore_barrier()
    g = c * NS + s
    pltpu.sync_copy(dst_hbm.at[pl.ds(g * EPT_D, EPT_D)], idxv)
    pltpu.sync_copy(onesv.at[pl.ds(0, EPT_D)], acc.at[idxv], add=True)
    plsc.subcore_barrier()
    pltpu.sync_copy(acc.at[pl.ds(s * 640, 640)],
                    out_hbm.at[pl.ds(c * ND + s * 640, 640)])


CH = 176
NCH = 57
EPAD = CH * NCH
NP = 10240
TRASH = NP - 1
RPT = NP // NS


@functools.partial(
    pl.kernel,
    out_type=jax.ShapeDtypeStruct((NC, NP, HH), jnp.float32),
    mesh=_mesh,
    scratch_types=[
        [pltpu.VMEM((CH,), jnp.int32)] * 2,
        [pltpu.VMEM((CH,), jnp.int32)] * 3,
        [pltpu.VMEM((CH, HH), jnp.float32)] * 2,
        pltpu.VMEM_SHARED((NP, HH), jnp.float32),
        pltpu.SemaphoreType.DMA,
        pltpu.SemaphoreType.DMA,
        pltpu.SemaphoreType.DMA,
    ],
)
def _sc_edge_pass(src2_hbm, dst_hbm, hs_hbm, z_hbm, out_hbm,
                  sv, dv, rows, acc, gsem, isem, ssem):
    c = lax.axis_index("c")
    s = lax.axis_index("s")

    def idx_start(g):
        return (
            pltpu.async_copy(
                src2_hbm.at[pl.ds((c * NS + s) * EPAD + g * CH, CH)],
                sv[g % 2], isem),
            pltpu.async_copy(
                dst_hbm.at[pl.ds(s * EPAD + g * CH, CH)], dv[g % 3], isem),
        )

    def gather_start(g):
        b = g % 2
        return pltpu.async_copy(hs_hbm.at[sv[b]], rows[b], gsem)

    h = idx_start(0)
    h[0].wait()
    h[1].wait()
    gcur = gather_start(0)
    inext = idx_start(1)
    pltpu.sync_copy(z_hbm, acc.at[pl.ds(s * RPT, RPT), :])
    plsc.subcore_barrier()
    sprev = None
    for g in range(NCH):
        gcur.wait()
        scur = pltpu.async_copy(rows[g % 2], acc.at[dv[g % 3]], ssem, add=True)
        if g + 1 < NCH:
            inext[0].wait()
            inext[1].wait()
            if sprev is not None:
                sprev.wait()
            gnext = gather_start(g + 1)
            gcur = gnext
        if g + 2 < NCH:
            inext = idx_start(g + 2)
        sprev = scur
    sprev.wait()
    if NCH > 1:
        sprev.wait()
    plsc.subcore_barrier()
    pltpu.sync_copy(acc.at[pl.ds(s * RPT, RPT), :],
                    out_hbm.at[c, pl.ds(s * RPT, RPT), :])


R = 1000
NB = N // R


def _elu(x):
    return jnp.where(x > 0, x, jnp.exp(jnp.minimum(x, 0.0)) - 1.0)


def _dinv_of(degp_ref):
    deg = degp_ref[:, 0:1] + degp_ref[:, 1:2] + 1.0
    return lax.rsqrt(deg)


def _tc1a_body(x_ref, w_ref, b_ref, x0_ref):
    x0 = jnp.dot(x_ref[...], w_ref[0], preferred_element_type=jnp.float32)
    x0_ref[...] = x0 + b_ref[0]


def _tc1b_body(x0_ref, degp_ref, wc0_ref, bc0_ref, hs_ref, sr1_ref):
    dinv = _dinv_of(degp_ref)
    h1 = jnp.dot(x0_ref[...], wc0_ref[...], preferred_element_type=jnp.float32)
    hs_ref[...] = h1 * dinv
    sr1_ref[...] = h1 * (dinv * dinv) + bc0_ref[...]


def _tc2_body(acca_ref, accb_ref, sr1_ref, h0_ref, degp_ref, wc1_ref, bc1_ref,
              hs_ref, sr2_ref):
    dinv = _dinv_of(degp_ref)
    acc = jnp.concatenate([acca_ref[0], accb_ref[0]], axis=1)
    x1 = _elu(acc * dinv + sr1_ref[...]) + h0_ref[...]
    h2 = jnp.dot(x1, wc1_ref[...], preferred_element_type=jnp.float32)
    hs_ref[...] = h2 * dinv
    sr2_ref[...] = h2 * (dinv * dinv) + bc1_ref[...]


def _tc3_body(acca_ref, accb_ref, sr2_ref, h0_ref, degp_ref, out_ref):
    dinv = _dinv_of(degp_ref)
    acc = jnp.concatenate([acca_ref[0], accb_ref[0]], axis=1)
    out_ref[...] = _elu(acc * dinv + sr2_ref[...]) + h0_ref[...]


def _rb(width):
    return pl.BlockSpec((R, width), lambda i: (i, 0))


def _acc_spec(core):
    return pl.BlockSpec((1, R, HH), lambda i, core=core: (core, i, 0))


def _full(shape):
    return pl.BlockSpec(shape, lambda i: tuple(0 for _ in shape))


_f32 = jnp.float32


def _tc1a(x_cat, w_stack, b_stack):
    return pl.pallas_call(
        _tc1a_body,
        grid=(NB,),
        in_specs=[
            _rb(H),
            pl.BlockSpec((1, H, H), lambda i: (i * R // (N // 2), 0, 0)),
            pl.BlockSpec((1, 1, H), lambda i: (i * R // (N // 2), 0, 0)),
        ],
        out_specs=_rb(H),
        out_shape=jax.ShapeDtypeStruct((N, H), _f32),
    )(x_cat, w_stack, b_stack)


def _tc1b(x0, degp, wc0, bc0):
    return pl.pallas_call(
        _tc1b_body,
        grid=(NB,),
        in_specs=[
            _rb(H),
            _rb(2),
            _full((H, H)),
            _full((1, H)),
        ],
        out_specs=[_rb(H), _rb(H)],
        out_shape=[
            jax.ShapeDtypeStruct((N, H), _f32),
            jax.ShapeDtypeStruct((N, H), _f32),
        ],
    )(x0, degp, wc0, bc0)


def _tc2(acc, sr1, h0, degp, wc1, bc1):
    return pl.pallas_call(
        _tc2_body,
        grid=(NB,),
        in_specs=[
            _acc_spec(0),
            _acc_spec(1),
            _rb(H),
            _rb(H),
            _rb(2),
            _full((H, H)),
            _full((1, H)),
        ],
        out_specs=[_rb(H), _rb(H)],
        out_shape=[
            jax.ShapeDtypeStruct((N, H), _f32),
            jax.ShapeDtypeStruct((N, H), _f32),
        ],
    )(acc, acc, sr1, h0, degp, wc1, bc1)


def _tc3(acc, sr2, h0, degp):
    return pl.pallas_call(
        _tc3_body,
        grid=(NB,),
        in_specs=[
            _acc_spec(0),
            _acc_spec(1),
            _rb(H),
            _rb(H),
            _rb(2),
        ],
        out_specs=_rb(H),
        out_shape=jax.ShapeDtypeStruct((N, H), _f32),
    )(acc, acc, sr2, h0, degp)


def kernel(x_lnc, x_dis, edge_index, W_lnc, b_lnc, W_dis, b_dis,
           W_c0, b_c0, W_c1, b_c1):
    src = edge_index[0]
    dst = edge_index[1]
    srcr = jnp.pad(src.reshape(NS, E // NS), ((0, 0), (0, EPAD - E // NS)))
    dstp = jnp.pad(dst.reshape(NS, E // NS), ((0, 0), (0, EPAD - E // NS)),
                   constant_values=TRASH).reshape(-1)
    src2 = jnp.stack([2 * srcr, 2 * srcr + 1]).reshape(-1)

    deg_flat = _sc_degree(dst)
    degp = deg_flat.reshape(NC, ND).transpose(1, 0)

    x_cat = jnp.concatenate([x_lnc, x_dis], axis=0)
    w_stack = jnp.stack([W_lnc, W_dis])
    b_stack = jnp.stack([b_lnc, b_dis]).reshape(2, 1, H)
    z_rows = jnp.zeros((RPT, HH), _f32)

    h0 = _tc1a(x_cat, w_stack, b_stack)
    hs1, sr1 = _tc1b(h0, degp, W_c0, b_c0.reshape(1, H))
    acc1 = _sc_edge_pass(src2, dstp, hs1.reshape(2 * N, HH), z_rows)

    hs2, sr2 = _tc2(acc1, sr1, h0, degp, W_c1, b_c1.reshape(1, H))
    acc2 = _sc_edge_pass(src2, dstp, hs2.reshape(2 * N, HH), z_rows)

    return _tc3(acc2, sr2, h0, degp)


# speedup vs baseline: 1.0224x; 1.0215x over previous
import functools

import jax
import jax.numpy as jnp
from jax import lax
from jax.experimental import pallas as pl
from jax.experimental.pallas import tpu as pltpu
from jax.experimental.pallas import tpu_sc as plsc

N = 10000
E = 160000
H = 256
HH = 128
NC, NS = 2, 16
ND = 10240

_mesh = plsc.VectorSubcoreMesh(core_axis_name="c", subcore_axis_name="s")


EPT_D = E // (NC * NS)


@functools.partial(
    pl.kernel,
    out_type=jax.ShapeDtypeStruct((NC * ND,), jnp.float32),
    mesh=_mesh,
    scratch_types=[
        pltpu.VMEM((EPT_D,), jnp.int32),
        pltpu.VMEM((5008,), jnp.float32),
        pltpu.VMEM((640,), jnp.float32),
        pltpu.VMEM_SHARED((ND,), jnp.float32),
    ],
)
def _sc_degree(dst_hbm, out_hbm, idxv, onesv, zv, acc):
    c = lax.axis_index("c")
    s = lax.axis_index("s")
    for i in range(313):
        onesv[pl.ds(i * 16, 16)] = jnp.ones((16,), jnp.float32)
    for i in range(40):
        zv[pl.ds(i * 16, 16)] = jnp.zeros((16,), jnp.float32)
    pltpu.sync_copy(zv, acc.at[pl.ds(s * 640, 640)])
    plsc.subcore_barrier()
    g = c * NS + s
    pltpu.sync_copy(dst_hbm.at[pl.ds(g * EPT_D, EPT_D)], idxv)
    pltpu.sync_copy(onesv.at[pl.ds(0, EPT_D)], acc.at[idxv], add=True)
    plsc.subcore_barrier()
    pltpu.sync_copy(acc.at[pl.ds(s * 640, 640)],
                    out_hbm.at[pl.ds(c * ND + s * 640, 640)])


CH = 176
NCH = 57
EPAD = CH * NCH
NP = 10240
TRASH = NP - 1
RPT = NP // NS


@functools.partial(
    pl.kernel,
    out_type=jax.ShapeDtypeStruct((NC, NP, HH), jnp.float32),
    mesh=_mesh,
    scratch_types=[
        [pltpu.VMEM((CH,), jnp.int32)] * 2,
        [pltpu.VMEM((CH,), jnp.int32)] * 3,
        [pltpu.VMEM((CH, HH), jnp.float32)] * 2,
        pltpu.VMEM_SHARED((NP, HH), jnp.float32),
        pltpu.SemaphoreType.DMA,
        pltpu.SemaphoreType.DMA,
        pltpu.SemaphoreType.DMA,
    ],
)
def _sc_edge_pass(src2_hbm, dst_hbm, hs_hbm, z_hbm, out_hbm,
                  sv, dv, rows, acc, gsem, isem, ssem):
    c = lax.axis_index("c")
    s = lax.axis_index("s")
    pltpu.sync_copy(z_hbm, acc.at[pl.ds(s * RPT, RPT), :])
    plsc.subcore_barrier()

    def idx_start(g):
        return (
            pltpu.async_copy(
                src2_hbm.at[pl.ds((c * NS + s) * EPAD + g * CH, CH)],
                sv[g % 2], isem),
            pltpu.async_copy(
                dst_hbm.at[pl.ds(s * EPAD + g * CH, CH)], dv[g % 3], isem),
        )

    def gather_start(g):
        b = g % 2
        return pltpu.async_copy(hs_hbm.at[sv[b]], rows[b], gsem)

    h = idx_start(0)
    h[0].wait()
    h[1].wait()
    gcur = gather_start(0)
    inext = idx_start(1)
    sprev = None
    for g in range(NCH):
        gcur.wait()
        scur = pltpu.async_copy(rows[g % 2], acc.at[dv[g % 3]], ssem, add=True)
        if g + 1 < NCH:
            inext[0].wait()
            inext[1].wait()
            if sprev is not None:
                sprev.wait()
            gnext = gather_start(g + 1)
            gcur = gnext
        if g + 2 < NCH:
            inext = idx_start(g + 2)
        sprev = scur
    sprev.wait()
    if NCH > 1:
        sprev.wait()
    plsc.subcore_barrier()
    pltpu.sync_copy(acc.at[pl.ds(s * RPT, RPT), :],
                    out_hbm.at[c, pl.ds(s * RPT, RPT), :])


R = 1000
NB = N // R


def _elu(x):
    return jnp.where(x > 0, x, jnp.exp(jnp.minimum(x, 0.0)) - 1.0)


def _dinv_of(degp_ref):
    deg = degp_ref[:, 0:1] + degp_ref[:, 1:2] + 1.0
    return lax.rsqrt(deg)


def _tc1_body(x_ref, w_ref, b_ref, degp_ref, wc0_ref, bc0_ref,
              x0_ref, hs_ref, sr1_ref):
    x0 = jnp.dot(x_ref[...], w_ref[0], preferred_element_type=jnp.float32)
    x0 = x0 + b_ref[0]
    x0_ref[...] = x0
    dinv = _dinv_of(degp_ref)
    h1 = jnp.dot(x0, wc0_ref[...], preferred_element_type=jnp.float32)
    hs_ref[...] = h1 * dinv
    sr1_ref[...] = h1 * (dinv * dinv) + bc0_ref[...]


def _tc2_body(acca_ref, accb_ref, sr1_ref, h0_ref, degp_ref, wc1_ref, bc1_ref,
              hs_ref, sr2_ref):
    dinv = _dinv_of(degp_ref)
    acc = jnp.concatenate([acca_ref[0], accb_ref[0]], axis=1)
    x1 = _elu(acc * dinv + sr1_ref[...]) + h0_ref[...]
    h2 = jnp.dot(x1, wc1_ref[...], preferred_element_type=jnp.float32)
    hs_ref[...] = h2 * dinv
    sr2_ref[...] = h2 * (dinv * dinv) + bc1_ref[...]


def _tc3_body(acca_ref, accb_ref, sr2_ref, h0_ref, degp_ref, out_ref):
    dinv = _dinv_of(degp_ref)
    acc = jnp.concatenate([acca_ref[0], accb_ref[0]], axis=1)
    out_ref[...] = _elu(acc * dinv + sr2_ref[...]) + h0_ref[...]


def _rb(width):
    return pl.BlockSpec((R, width), lambda i: (i, 0))


def _acc_spec(core):
    return pl.BlockSpec((1, R, HH), lambda i, core=core: (core, i, 0))


def _full(shape):
    return pl.BlockSpec(shape, lambda i: tuple(0 for _ in shape))


_f32 = jnp.float32


def _tc1(x_cat, w_stack, b_stack, degp, wc0, bc0):
    return pl.pallas_call(
        _tc1_body,
        grid=(NB,),
        in_specs=[
            _rb(H),
            pl.BlockSpec((1, H, H), lambda i: (i * R // (N // 2), 0, 0)),
            pl.BlockSpec((1, 1, H), lambda i: (i * R // (N // 2), 0, 0)),
            _rb(2),
            _full((H, H)),
            _full((1, H)),
        ],
        out_specs=[_rb(H), _rb(H), _rb(H)],
        out_shape=[
            jax.ShapeDtypeStruct((N, H), _f32),
            jax.ShapeDtypeStruct((N, H), _f32),
            jax.ShapeDtypeStruct((N, H), _f32),
        ],
    )(x_cat, w_stack, b_stack, degp, wc0, bc0)


def _tc2(acc, sr1, h0, degp, wc1, bc1):
    return pl.pallas_call(
        _tc2_body,
        grid=(NB,),
        in_specs=[
            _acc_spec(0),
            _acc_spec(1),
            _rb(H),
            _rb(H),
            _rb(2),
            _full((H, H)),
            _full((1, H)),
        ],
        out_specs=[_rb(H), _rb(H)],
        out_shape=[
            jax.ShapeDtypeStruct((N, H), _f32),
            jax.ShapeDtypeStruct((N, H), _f32),
        ],
    )(acc, acc, sr1, h0, degp, wc1, bc1)


def _tc3(acc, sr2, h0, degp):
    return pl.pallas_call(
        _tc3_body,
        grid=(NB,),
        in_specs=[
            _acc_spec(0),
            _acc_spec(1),
            _rb(H),
            _rb(H),
            _rb(2),
        ],
        out_specs=_rb(H),
        out_shape=jax.ShapeDtypeStruct((N, H), _f32),
    )(acc, acc, sr2, h0, degp)


def kernel(x_lnc, x_dis, edge_index, W_lnc, b_lnc, W_dis, b_dis,
           W_c0, b_c0, W_c1, b_c1):
    src = edge_index[0]
    dst = edge_index[1]
    srcr = jnp.pad(src.reshape(NS, E // NS), ((0, 0), (0, EPAD - E // NS)))
    dstp = jnp.pad(dst.reshape(NS, E // NS), ((0, 0), (0, EPAD - E // NS)),
                   constant_values=TRASH).reshape(-1)
    src2 = jnp.stack([2 * srcr, 2 * srcr + 1]).reshape(-1)

    deg_flat = _sc_degree(dst)
    degp = deg_flat.reshape(NC, ND).transpose(1, 0)

    x_cat = jnp.concatenate([x_lnc, x_dis], axis=0)
    w_stack = jnp.stack([W_lnc, W_dis])
    b_stack = jnp.stack([b_lnc, b_dis]).reshape(2, 1, H)
    z_rows = jnp.zeros((RPT, HH), _f32)

    h0, hs1, sr1 = _tc1(x_cat, w_stack, b_stack, degp,
                        W_c0, b_c0.reshape(1, H))
    acc1 = _sc_edge_pass(src2, dstp, hs1.reshape(2 * N, HH), z_rows)

    hs2, sr2 = _tc2(acc1, sr1, h0, degp, W_c1, b_c1.reshape(1, H))
    acc2 = _sc_edge_pass(src2, dstp, hs2.reshape(2 * N, HH), z_rows)

    return _tc3(acc2, sr2, h0, degp)


# speedup vs baseline: 1.0246x; 1.0021x over previous
import functools

import jax
import jax.numpy as jnp
from jax import lax
from jax.experimental import pallas as pl
from jax.experimental.pallas import tpu as pltpu
from jax.experimental.pallas import tpu_sc as plsc

N = 10000
E = 160000
H = 256
HH = 128
NC, NS = 2, 16
ND = 10240

_mesh = plsc.VectorSubcoreMesh(core_axis_name="c", subcore_axis_name="s")


EPT_D = E // (NC * NS)


@functools.partial(
    pl.kernel,
    out_type=jax.ShapeDtypeStruct((NC * ND,), jnp.float32),
    mesh=_mesh,
    scratch_types=[
        pltpu.VMEM((EPT_D,), jnp.int32),
        pltpu.VMEM((5008,), jnp.float32),
        pltpu.VMEM((640,), jnp.float32),
        pltpu.VMEM_SHARED((ND,), jnp.float32),
    ],
)
def _sc_degree(dst_hbm, out_hbm, idxv, onesv, zv, acc):
    c = lax.axis_index("c")
    s = lax.axis_index("s")
    for i in range(313):
        onesv[pl.ds(i * 16, 16)] = jnp.ones((16,), jnp.float32)
    for i in range(40):
        zv[pl.ds(i * 16, 16)] = jnp.zeros((16,), jnp.float32)
    pltpu.sync_copy(zv, acc.at[pl.ds(s * 640, 640)])
    plsc.subcore_barrier()
    g = c * NS + s
    pltpu.sync_copy(dst_hbm.at[pl.ds(g * EPT_D, EPT_D)], idxv)
    pltpu.sync_copy(onesv.at[pl.ds(0, EPT_D)], acc.at[idxv], add=True)
    plsc.subcore_barrier()
    pltpu.sync_copy(acc.at[pl.ds(s * 640, 640)],
                    out_hbm.at[pl.ds(c * ND + s * 640, 640)])


CH = 176
NCH = 57
EPAD = CH * NCH
NP = 10240
TRASH = NP - 1
RPT = NP // NS


@functools.partial(
    pl.kernel,
    out_type=jax.ShapeDtypeStruct((NC, NP, HH), jnp.float32),
    mesh=_mesh,
    scratch_types=[
        [pltpu.VMEM((CH,), jnp.int32)] * 2,
        [pltpu.VMEM((CH,), jnp.int32)] * 3,
        [pltpu.VMEM((CH, HH), jnp.float32)] * 2,
        pltpu.VMEM_SHARED((NP, HH), jnp.float32),
        pltpu.SemaphoreType.DMA,
        pltpu.SemaphoreType.DMA,
        pltpu.SemaphoreType.DMA,
    ],
)
def _sc_edge_pass(src2_hbm, dst_hbm, hs_hbm, z_hbm, out_hbm,
                  sv, dv, rows, acc, gsem, isem, ssem):
    c = lax.axis_index("c")
    s = lax.axis_index("s")

    def idx_start(g):
        return (
            pltpu.async_copy(
                src2_hbm.at[pl.ds((c * NS + s) * EPAD + g * CH, CH)],
                sv[g % 2], isem),
            pltpu.async_copy(
                dst_hbm.at[pl.ds(s * EPAD + g * CH, CH)], dv[g % 3], isem),
        )

    def gather_start(g):
        b = g % 2
        return pltpu.async_copy(hs_hbm.at[sv[b]], rows[b], gsem)

    h = idx_start(0)
    h[0].wait()
    h[1].wait()
    gcur = gather_start(0)
    inext = idx_start(1)
    pltpu.sync_copy(z_hbm, acc.at[pl.ds(s * RPT, RPT), :])
    plsc.subcore_barrier()
    sprev = None
    for g in range(NCH):
        gcur.wait()
        scur = pltpu.async_copy(rows[g % 2], acc.at[dv[g % 3]], ssem, add=True)
        if g + 1 < NCH:
            inext[0].wait()
            inext[1].wait()
            if sprev is not None:
                sprev.wait()
            gnext = gather_start(g + 1)
            gcur = gnext
        if g + 2 < NCH:
            inext = idx_start(g + 2)
        sprev = scur
    sprev.wait()
    if NCH > 1:
        sprev.wait()
    plsc.subcore_barrier()
    pltpu.sync_copy(acc.at[pl.ds(s * RPT, RPT), :],
                    out_hbm.at[c, pl.ds(s * RPT, RPT), :])


R = 1000
NB = N // R


def _elu(x):
    return jnp.where(x > 0, x, jnp.exp(jnp.minimum(x, 0.0)) - 1.0)


def _dinv_of(degp_ref):
    deg = degp_ref[:, 0:1] + degp_ref[:, 1:2] + 1.0
    return lax.rsqrt(deg)


def _tc1_body(x_ref, w_ref, b_ref, degp_ref, wc0_ref, bc0_ref,
              x0_ref, hs_ref, sr1_ref):
    x0 = jnp.dot(x_ref[...], w_ref[0], preferred_element_type=jnp.float32)
    x0 = x0 + b_ref[0]
    x0_ref[...] = x0
    dinv = _dinv_of(degp_ref)
    h1 = jnp.dot(x0, wc0_ref[...], preferred_element_type=jnp.float32)
    hs_ref[...] = h1 * dinv
    sr1_ref[...] = h1 * (dinv * dinv) + bc0_ref[...]


def _tc2_body(acca_ref, accb_ref, sr1_ref, h0_ref, degp_ref, wc1_ref, bc1_ref,
              hs_ref, sr2_ref):
    dinv = _dinv_of(degp_ref)
    acc = jnp.concatenate([acca_ref[0], accb_ref[0]], axis=1)
    x1 = _elu(acc * dinv + sr1_ref[...]) + h0_ref[...]
    h2 = jnp.dot(x1, wc1_ref[...], preferred_element_type=jnp.float32)
    hs_ref[...] = h2 * dinv
    sr2_ref[...] = h2 * (dinv * dinv) + bc1_ref[...]


def _tc3_body(acca_ref, accb_ref, sr2_ref, h0_ref, degp_ref, out_ref):
    dinv = _dinv_of(degp_ref)
    acc = jnp.concatenate([acca_ref[0], accb_ref[0]], axis=1)
    out_ref[...] = _elu(acc * dinv + sr2_ref[...]) + h0_ref[...]


def _rb(width):
    return pl.BlockSpec((R, width), lambda i: (i, 0))


def _acc_spec(core):
    return pl.BlockSpec((1, R, HH), lambda i, core=core: (core, i, 0))


def _full(shape):
    return pl.BlockSpec(shape, lambda i: tuple(0 for _ in shape))


_f32 = jnp.float32


def _tc1(x_cat, w_stack, b_stack, degp, wc0, bc0):
    return pl.pallas_call(
        _tc1_body,
        grid=(NB,),
        in_specs=[
            _rb(H),
            pl.BlockSpec((1, H, H), lambda i: (i * R // (N // 2), 0, 0)),
            pl.BlockSpec((1, 1, H), lambda i: (i * R // (N // 2), 0, 0)),
            _rb(2),
            _full((H, H)),
            _full((1, H)),
        ],
        out_specs=[_rb(H), _rb(H), _rb(H)],
        out_shape=[
            jax.ShapeDtypeStruct((N, H), _f32),
            jax.ShapeDtypeStruct((N, H), _f32),
            jax.ShapeDtypeStruct((N, H), _f32),
        ],
    )(x_cat, w_stack, b_stack, degp, wc0, bc0)


def _tc2(acc, sr1, h0, degp, wc1, bc1):
    return pl.pallas_call(
        _tc2_body,
        grid=(NB,),
        in_specs=[
            _acc_spec(0),
            _acc_spec(1),
            _rb(H),
            _rb(H),
            _rb(2),
            _full((H, H)),
            _full((1, H)),
        ],
        out_specs=[_rb(H), _rb(H)],
        out_shape=[
            jax.ShapeDtypeStruct((N, H), _f32),
            jax.ShapeDtypeStruct((N, H), _f32),
        ],
    )(acc, acc, sr1, h0, degp, wc1, bc1)


def _tc3(acc, sr2, h0, degp):
    return pl.pallas_call(
        _tc3_body,
        grid=(NB,),
        in_specs=[
            _acc_spec(0),
            _acc_spec(1),
            _rb(H),
            _rb(H),
            _rb(2),
        ],
        out_specs=_rb(H),
        out_shape=jax.ShapeDtypeStruct((N, H), _f32),
    )(acc, acc, sr2, h0, degp)


def kernel(x_lnc, x_dis, edge_index, W_lnc, b_lnc, W_dis, b_dis,
           W_c0, b_c0, W_c1, b_c1):
    src = edge_index[0]
    dst = edge_index[1]
    srcr = jnp.pad(src.reshape(NS, E // NS), ((0, 0), (0, EPAD - E // NS)))
    dstp = jnp.pad(dst.reshape(NS, E // NS), ((0, 0), (0, EPAD - E // NS)),
                   constant_values=TRASH).reshape(-1)
    src2 = jnp.stack([2 * srcr, 2 * srcr + 1]).reshape(-1)

    deg_flat = _sc_degree(dst)
    degp = deg_flat.reshape(NC, ND).transpose(1, 0)

    x_cat = jnp.concatenate([x_lnc, x_dis], axis=0)
    w_stack = jnp.stack([W_lnc, W_dis])
    b_stack = jnp.stack([b_lnc, b_dis]).reshape(2, 1, H)
    z_rows = jnp.zeros((RPT, HH), _f32)

    h0, hs1, sr1 = _tc1(x_cat, w_stack, b_stack, degp,
                        W_c0, b_c0.reshape(1, H))
    acc1 = _sc_edge_pass(src2, dstp, hs1.reshape(2 * N, HH), z_rows)

    hs2, sr2 = _tc2(acc1, sr1, h0, degp, W_c1, b_c1.reshape(1, H))
    acc2 = _sc_edge_pass(src2, dstp, hs2.reshape(2 * N, HH), z_rows)

    return _tc3(acc2, sr2, h0, degp)


# speedup vs baseline: 1.0254x; 1.0008x over previous
import functools

import jax
import jax.numpy as jnp
from jax import lax
from jax.experimental import pallas as pl
from jax.experimental.pallas import tpu as pltpu
from jax.experimental.pallas import tpu_sc as plsc

N = 10000
E = 160000
H = 256
HH = 128
NC, NS = 2, 16
ND = 10240

_mesh = plsc.VectorSubcoreMesh(core_axis_name="c", subcore_axis_name="s")


EPT_D = E // (NC * NS)


@functools.partial(
    pl.kernel,
    out_type=jax.ShapeDtypeStruct((NC * ND,), jnp.float32),
    mesh=_mesh,
    scratch_types=[
        pltpu.VMEM((EPT_D,), jnp.int32),
        pltpu.VMEM((5008,), jnp.float32),
        pltpu.VMEM((640,), jnp.float32),
        pltpu.VMEM_SHARED((ND,), jnp.float32),
    ],
)
def _sc_degree(dst_hbm, out_hbm, idxv, onesv, zv, acc):
    c = lax.axis_index("c")
    s = lax.axis_index("s")
    for i in range(313):
        onesv[pl.ds(i * 16, 16)] = jnp.ones((16,), jnp.float32)
    for i in range(40):
        zv[pl.ds(i * 16, 16)] = jnp.zeros((16,), jnp.float32)
    pltpu.sync_copy(zv, acc.at[pl.ds(s * 640, 640)])
    plsc.subcore_barrier()
    g = c * NS + s
    pltpu.sync_copy(dst_hbm.at[pl.ds(g * EPT_D, EPT_D)], idxv)
    pltpu.sync_copy(onesv.at[pl.ds(0, EPT_D)], acc.at[idxv], add=True)
    plsc.subcore_barrier()
    pltpu.sync_copy(acc.at[pl.ds(s * 640, 640)],
                    out_hbm.at[pl.ds(c * ND + s * 640, 640)])


CH = 176
NCH = 57
EPAD = CH * NCH
NP = 10240
TRASH = NP - 1
RPT = NP // NS


@functools.partial(
    pl.kernel,
    out_type=jax.ShapeDtypeStruct((NC, NP, HH), jnp.float32),
    mesh=_mesh,
    scratch_types=[
        [pltpu.VMEM((CH,), jnp.int32)] * 2,
        [pltpu.VMEM((CH,), jnp.int32)] * 3,
        [pltpu.VMEM((CH, HH), jnp.float32)] * 2,
        pltpu.VMEM_SHARED((NP, HH), jnp.float32),
        pltpu.SemaphoreType.DMA,
        pltpu.SemaphoreType.DMA,
    ],
)
def _sc_edge_pass(src2_hbm, dst_hbm, hs_hbm, z_hbm, out_hbm,
                  sv, dv, rows, acc, gsem, isem):
    c = lax.axis_index("c")
    s = lax.axis_index("s")

    def idx_start(g):
        return (
            pltpu.async_copy(
                src2_hbm.at[pl.ds((c * NS + s) * EPAD + g * CH, CH)],
                sv[g % 2], isem),
            pltpu.async_copy(
                dst_hbm.at[pl.ds(s * EPAD + g * CH, CH)], dv[g % 3], isem),
        )

    def gather_start(g):
        b = g % 2
        return pltpu.async_copy(hs_hbm.at[sv[b]], rows[b], gsem)

    h = idx_start(0)
    h[0].wait()
    h[1].wait()
    gcur = gather_start(0)
    inext = idx_start(1)
    pltpu.sync_copy(z_hbm, acc.at[pl.ds(s * RPT, RPT), :])
    plsc.subcore_barrier()
    for g in range(NCH):
        gcur.wait()
        if g + 1 < NCH:
            inext[0].wait()
            inext[1].wait()
            gnext = gather_start(g + 1)
        pltpu.sync_copy(rows[g % 2], acc.at[dv[g % 3]], add=True)
        if g + 2 < NCH:
            inext = idx_start(g + 2)
        if g + 1 < NCH:
            gcur = gnext
    plsc.subcore_barrier()
    pltpu.sync_copy(acc.at[pl.ds(s * RPT, RPT), :],
                    out_hbm.at[c, pl.ds(s * RPT, RPT), :])


R = 1000
NB = N // R


def _elu(x):
    return jnp.where(x > 0, x, jnp.exp(jnp.minimum(x, 0.0)) - 1.0)


def _dinv_of(degp_ref):
    deg = degp_ref[:, 0:1] + degp_ref[:, 1:2] + 1.0
    return lax.rsqrt(deg)


def _tc1_body(x_ref, w_ref, b_ref, degp_ref, wc0_ref, bc0_ref,
              x0_ref, hs_ref, sr1_ref):
    x0 = jnp.dot(x_ref[...], w_ref[0], preferred_element_type=jnp.float32)
    x0 = x0 + b_ref[0]
    x0_ref[...] = x0
    dinv = _dinv_of(degp_ref)
    h1 = jnp.dot(x0, wc0_ref[...], preferred_element_type=jnp.float32)
    hs_ref[...] = h1 * dinv
    sr1_ref[...] = h1 * (dinv * dinv) + bc0_ref[...]


def _tc2_body(acca_ref, accb_ref, sr1_ref, h0_ref, degp_ref, wc1_ref, bc1_ref,
              hs_ref, sr2_ref):
    dinv = _dinv_of(degp_ref)
    acc = jnp.concatenate([acca_ref[0], accb_ref[0]], axis=1)
    x1 = _elu(acc * dinv + sr1_ref[...]) + h0_ref[...]
    h2 = jnp.dot(x1, wc1_ref[...], preferred_element_type=jnp.float32)
    hs_ref[...] = h2 * dinv
    sr2_ref[...] = h2 * (dinv * dinv) + bc1_ref[...]


def _tc3_body(acca_ref, accb_ref, sr2_ref, h0_ref, degp_ref, out_ref):
    dinv = _dinv_of(degp_ref)
    acc = jnp.concatenate([acca_ref[0], accb_ref[0]], axis=1)
    out_ref[...] = _elu(acc * dinv + sr2_ref[...]) + h0_ref[...]


def _rb(width):
    return pl.BlockSpec((R, width), lambda i: (i, 0))


def _acc_spec(core):
    return pl.BlockSpec((1, R, HH), lambda i, core=core: (core, i, 0))


def _full(shape):
    return pl.BlockSpec(shape, lambda i: tuple(0 for _ in shape))


_f32 = jnp.float32


def _tc1(x_cat, w_stack, b_stack, degp, wc0, bc0):
    return pl.pallas_call(
        _tc1_body,
        grid=(NB,),
        in_specs=[
            _rb(H),
            pl.BlockSpec((1, H, H), lambda i: (i * R // (N // 2), 0, 0)),
            pl.BlockSpec((1, 1, H), lambda i: (i * R // (N // 2), 0, 0)),
            _rb(2),
            _full((H, H)),
            _full((1, H)),
        ],
        out_specs=[_rb(H), _rb(H), _rb(H)],
        out_shape=[
            jax.ShapeDtypeStruct((N, H), _f32),
            jax.ShapeDtypeStruct((N, H), _f32),
            jax.ShapeDtypeStruct((N, H), _f32),
        ],
    )(x_cat, w_stack, b_stack, degp, wc0, bc0)


def _tc2(acc, sr1, h0, degp, wc1, bc1):
    return pl.pallas_call(
        _tc2_body,
        grid=(NB,),
        in_specs=[
            _acc_spec(0),
            _acc_spec(1),
            _rb(H),
            _rb(H),
            _rb(2),
            _full((H, H)),
            _full((1, H)),
        ],
        out_specs=[_rb(H), _rb(H)],
        out_shape=[
            jax.ShapeDtypeStruct((N, H), _f32),
            jax.ShapeDtypeStruct((N, H), _f32),
        ],
    )(acc, acc, sr1, h0, degp, wc1, bc1)


def _tc3(acc, sr2, h0, degp):
    return pl.pallas_call(
        _tc3_body,
        grid=(NB,),
        in_specs=[
            _acc_spec(0),
            _acc_spec(1),
            _rb(H),
            _rb(H),
            _rb(2),
        ],
        out_specs=_rb(H),
        out_shape=jax.ShapeDtypeStruct((N, H), _f32),
    )(acc, acc, sr2, h0, degp)


def kernel(x_lnc, x_dis, edge_index, W_lnc, b_lnc, W_dis, b_dis,
           W_c0, b_c0, W_c1, b_c1):
    src = edge_index[0]
    dst = edge_index[1]
    srcr = jnp.pad(src.reshape(NS, E // NS), ((0, 0), (0, EPAD - E // NS)))
    dstp = jnp.pad(dst.reshape(NS, E // NS), ((0, 0), (0, EPAD - E // NS)),
                   constant_values=TRASH).reshape(-1)
    src2 = jnp.stack([2 * srcr, 2 * srcr + 1]).reshape(-1)

    deg_flat = _sc_degree(dst)
    degp = deg_flat.reshape(NC, ND).transpose(1, 0)

    x_cat = jnp.concatenate([x_lnc, x_dis], axis=0)
    w_stack = jnp.stack([W_lnc, W_dis])
    b_stack = jnp.stack([b_lnc, b_dis]).reshape(2, 1, H)
    z_rows = jnp.zeros((RPT, HH), _f32)

    h0, hs1, sr1 = _tc1(x_cat, w_stack, b_stack, degp,
                        W_c0, b_c0.reshape(1, H))
    acc1 = _sc_edge_pass(src2, dstp, hs1.reshape(2 * N, HH), z_rows)

    hs2, sr2 = _tc2(acc1, sr1, h0, degp, W_c1, b_c1.reshape(1, H))
    acc2 = _sc_edge_pass(src2, dstp, hs2.reshape(2 * N, HH), z_rows)

    return _tc3(acc2, sr2, h0, degp)
